# baseline (speedup 1.0000x reference)
import functools

import jax
import jax.numpy as jnp
from jax import lax
from jax.experimental import pallas as pl
from jax.experimental.pallas import tpu as pltpu
from jax.experimental.pallas import tpu_sc as plsc

D_MODEL = 1024
N_BATCH = 4
SEQ = 4096
N_TOK = N_BATCH * SEQ
N_WORKERS = 32
S_PER_W = SEQ // N_WORKERS
CS = 4
ROWS = N_BATCH * CS
N_CHUNKS = S_PER_W // CS
NB = 4
LANES = 16


@functools.partial(
    pl.kernel,
    mesh=plsc.VectorSubcoreMesh(core_axis_name="c", subcore_axis_name="s"),
    out_type=jax.ShapeDtypeStruct((N_TOK, D_MODEL), jnp.float32),
    scratch_types=(
        [pltpu.VMEM((N_BATCH * S_PER_W,), jnp.int32)]
        + [pltpu.VMEM((ROWS, D_MODEL), jnp.float32) for _ in range(NB)]
        + [pltpu.VMEM((CS, D_MODEL), jnp.float32) for _ in range(NB)]
        + [pltpu.VMEM((ROWS, D_MODEL), jnp.float32) for _ in range(2)]
        + [pltpu.SemaphoreType.DMA for _ in range(2 * NB + 2)]
    ),
    compiler_params=pltpu.CompilerParams(
        needs_layout_passes=False,
        disable_bounds_checks=True,
        disable_semaphore_checks=True,
        skip_device_barrier=True,
    ),
)
def _embed_sc(x_hbm, table_hbm, pos_hbm, out_hbm,
              stage_v,
              rows0, rows1, rows2, rows3,
              pos0, pos1, pos2, pos3,
              sb0, sb1,
              g0, g1, g2, g3, p0, p1, p2, p3, o0, o1):
    rows_b = (rows0, rows1, rows2, rows3)
    pos_b = (pos0, pos1, pos2, pos3)
    g_sem = (g0, g1, g2, g3)
    p_sem = (p0, p1, p2, p3)
    sb = (sb0, sb1)
    o_sem = (o0, o1)

    wid = lax.axis_index("s") * 2 + lax.axis_index("c")
    s_base = wid * S_PER_W

    for bi in range(N_BATCH):
        pltpu.sync_copy(x_hbm.at[pl.ds(bi * SEQ + s_base, S_PER_W)],
                        stage_v.at[pl.ds(bi * S_PER_W, S_PER_W)])

    io = lax.iota(jnp.int32, LANES)
    lane_off = lax.shift_right_logical(io, 2) * S_PER_W + jnp.bitwise_and(io, 3)

    def issue_chunk(c, slot):
        ids = plsc.load_gather(stage_v, [lane_off + c * CS])
        pltpu.async_copy(table_hbm.at[ids], rows_b[slot], g_sem[slot])
        pltpu.async_copy(pos_hbm.at[pl.ds(s_base + c * CS, CS)],
                         pos_b[slot], p_sem[slot])

    for c in range(NB):
        issue_chunk(c, c)

    def outer(i, carry):
        for b in range(NB):
            c = i * NB + b
            sbi = b % 2
            @pl.when(c >= 2)
            def _():
                for bi in range(N_BATCH):
                    pltpu.make_async_copy(
                        sb[sbi].at[pl.ds(bi * CS, CS)],
                        out_hbm.at[pl.ds(bi * SEQ, CS)],
                        o_sem[sbi]).wait()
            pltpu.make_async_copy(table_hbm.at[io], rows_b[b],
                                  g_sem[b]).wait()
            pltpu.make_async_copy(pos_hbm.at[pl.ds(0, CS)], pos_b[b],
                                  p_sem[b]).wait()

            def add_body(j, jcarry):
                sl = pl.ds(j * LANES, LANES)
                for s in range(CS):
                    pv = pos_b[b][s, sl]
                    for bi in range(N_BATCH):
                        r = bi * CS + s
                        sb[sbi][r, sl] = rows_b[b][r, sl] + pv
                return jcarry
            lax.fori_loop(0, D_MODEL // LANES, add_body, 0)

            out_row = s_base + c * CS
            for bi in range(N_BATCH):
                pltpu.async_copy(sb[sbi].at[pl.ds(bi * CS, CS)],
                                 out_hbm.at[pl.ds(bi * SEQ + out_row, CS)],
                                 o_sem[sbi])

            @pl.when(c + NB < N_CHUNKS)
            def _():
                issue_chunk(c + NB, b)
        return carry

    lax.fori_loop(0, N_CHUNKS // NB, outer, 0)

    for sbi in range(2):
        for bi in range(N_BATCH):
            pltpu.make_async_copy(
                sb[sbi].at[pl.ds(bi * CS, CS)],
                out_hbm.at[pl.ds(bi * SEQ, CS)],
                o_sem[sbi]).wait()


def kernel(x, table, pos_encoding):
    out = _embed_sc(x.reshape(-1).astype(jnp.int32), table, pos_encoding)
    return out.reshape(N_BATCH, SEQ, D_MODEL)


# speedup vs baseline: 1.0174x; 1.0174x over previous
import functools

import jax
import jax.numpy as jnp
from jax import lax
from jax.experimental import pallas as pl
from jax.experimental.pallas import tpu as pltpu
from jax.experimental.pallas import tpu_sc as plsc

D_MODEL = 1024
N_BATCH = 4
SEQ = 4096
N_TOK = N_BATCH * SEQ
N_WORKERS = 32
S_PER_W = SEQ // N_WORKERS
CS = 4
ROWS = N_BATCH * CS
N_CHUNKS = S_PER_W // CS
NB = 4
LANES = 16


@functools.partial(
    pl.kernel,
    mesh=plsc.VectorSubcoreMesh(core_axis_name="c", subcore_axis_name="s"),
    out_type=jax.ShapeDtypeStruct((N_TOK, D_MODEL), jnp.float32),
    scratch_types=(
        [pltpu.VMEM((N_BATCH * S_PER_W,), jnp.int32)]
        + [pltpu.VMEM((ROWS, D_MODEL), jnp.float32) for _ in range(NB)]
        + [pltpu.VMEM((CS, D_MODEL), jnp.float32) for _ in range(NB)]
        + [pltpu.VMEM((ROWS, D_MODEL), jnp.float32) for _ in range(2)]
        + [pltpu.SemaphoreType.DMA for _ in range(2 * NB + 2)]
    ),
    compiler_params=pltpu.CompilerParams(needs_layout_passes=False),
)
def _embed_sc(x_hbm, table_hbm, pos_hbm, out_hbm,
              stage_v,
              rows0, rows1, rows2, rows3,
              pos0, pos1, pos2, pos3,
              sb0, sb1,
              g0, g1, g2, g3, p0, p1, p2, p3, o0, o1):
    rows_b = (rows0, rows1, rows2, rows3)
    pos_b = (pos0, pos1, pos2, pos3)
    g_sem = (g0, g1, g2, g3)
    p_sem = (p0, p1, p2, p3)
    sb = (sb0, sb1)
    o_sem = (o0, o1)

    wid = lax.axis_index("s") * 2 + lax.axis_index("c")
    s_base = wid * S_PER_W

    for bi in range(N_BATCH):
        pltpu.sync_copy(x_hbm.at[pl.ds(bi * SEQ + s_base, S_PER_W)],
                        stage_v.at[pl.ds(bi * S_PER_W, S_PER_W)])

    io = lax.iota(jnp.int32, LANES)
    lane_b = lax.shift_right_logical(io, 2)
    lane_s = jnp.bitwise_and(io, 3)
    lane_off = lane_b * S_PER_W + lane_s
    out_lane = lane_b * SEQ + s_base + lane_s

    def issue_chunk(c, slot):
        ids = plsc.load_gather(stage_v, [lane_off + c * CS])
        pltpu.async_copy(table_hbm.at[ids], rows_b[slot], g_sem[slot])
        pltpu.async_copy(pos_hbm.at[pl.ds(s_base + c * CS, CS)],
                         pos_b[slot], p_sem[slot])

    for c in range(NB):
        issue_chunk(c, c)

    def outer(i, carry):
        for b in range(NB):
            c = i * NB + b
            sbi = b % 2
            @pl.when(c >= 2)
            def _():
                pltpu.make_async_copy(sb[sbi], out_hbm.at[io],
                                      o_sem[sbi]).wait()
            pltpu.make_async_copy(table_hbm.at[io], rows_b[b],
                                  g_sem[b]).wait()
            pltpu.make_async_copy(pos_hbm.at[pl.ds(0, CS)], pos_b[b],
                                  p_sem[b]).wait()

            def add_body(j, jcarry):
                sl = pl.ds(j * LANES, LANES)
                for s in range(CS):
                    pv = pos_b[b][s, sl]
                    for bi in range(N_BATCH):
                        r = bi * CS + s
                        sb[sbi][r, sl] = rows_b[b][r, sl] + pv
                return jcarry
            lax.fori_loop(0, D_MODEL // LANES, add_body, 0)

            pltpu.async_copy(sb[sbi], out_hbm.at[out_lane + c * CS],
                             o_sem[sbi])

            @pl.when(c + NB < N_CHUNKS)
            def _():
                issue_chunk(c + NB, b)
        return carry

    lax.fori_loop(0, N_CHUNKS // NB, outer, 0)

    for sbi in range(2):
        pltpu.make_async_copy(sb[sbi], out_hbm.at[io], o_sem[sbi]).wait()


def kernel(x, table, pos_encoding):
    out = _embed_sc(x.reshape(-1).astype(jnp.int32), table, pos_encoding)
    return out.reshape(N_BATCH, SEQ, D_MODEL)


# speedup vs baseline: 1.0261x; 1.0086x over previous
import functools

import jax
import jax.numpy as jnp
from jax import lax
from jax.experimental import pallas as pl
from jax.experimental.pallas import tpu as pltpu
from jax.experimental.pallas import tpu_sc as plsc

D_MODEL = 1024
N_BATCH = 4
SEQ = 4096
N_TOK = N_BATCH * SEQ
N_WORKERS = 32
S_PER_W = SEQ // N_WORKERS
CS = 4
ROWS = N_BATCH * CS
N_CHUNKS = S_PER_W // CS
NB = 4
LANES = 16


@functools.partial(
    pl.kernel,
    mesh=plsc.VectorSubcoreMesh(core_axis_name="c", subcore_axis_name="s"),
    out_type=jax.ShapeDtypeStruct((N_TOK, D_MODEL), jnp.float32),
    scratch_types=(
        [pltpu.VMEM((N_BATCH * S_PER_W,), jnp.int32)]
        + [pltpu.VMEM((ROWS, D_MODEL), jnp.float32) for _ in range(NB)]
        + [pltpu.VMEM((CS, D_MODEL), jnp.float32) for _ in range(NB)]
        + [pltpu.VMEM((ROWS, D_MODEL), jnp.float32) for _ in range(2)]
        + [pltpu.SemaphoreType.DMA for _ in range(2 * NB + 2)]
    ),
    compiler_params=pltpu.CompilerParams(needs_layout_passes=False),
)
def _embed_sc(x_hbm, table_hbm, pos_hbm, out_hbm,
              stage_v,
              rows0, rows1, rows2, rows3,
              pos0, pos1, pos2, pos3,
              sb0, sb1,
              g0, g1, g2, g3, p0, p1, p2, p3, o0, o1):
    rows_b = (rows0, rows1, rows2, rows3)
    pos_b = (pos0, pos1, pos2, pos3)
    g_sem = (g0, g1, g2, g3)
    p_sem = (p0, p1, p2, p3)
    sb = (sb0, sb1)
    o_sem = (o0, o1)

    wid = lax.axis_index("s") * 2 + lax.axis_index("c")
    s_base = wid * S_PER_W

    io = lax.iota(jnp.int32, LANES)
    lane_b = lax.shift_right_logical(io, 2)
    lane_s = jnp.bitwise_and(io, 3)
    lane_off = lane_b * S_PER_W + lane_s
    out_lane = lane_b * SEQ + s_base + lane_s

    def issue_pos(c, slot):
        pltpu.async_copy(pos_hbm.at[pl.ds(s_base + c * CS, CS)],
                         pos_b[slot], p_sem[slot])

    def issue_gather(c, slot):
        ids = plsc.load_gather(stage_v, [lane_off + c * CS])
        pltpu.async_copy(table_hbm.at[ids], rows_b[slot], g_sem[slot])

    def issue_chunk(c, slot):
        issue_gather(c, slot)
        issue_pos(c, slot)

    for c in range(NB):
        issue_pos(c, c)
    stage_cps = [
        pltpu.async_copy(x_hbm.at[pl.ds(bi * SEQ + s_base, S_PER_W)],
                         stage_v.at[pl.ds(bi * S_PER_W, S_PER_W)],
                         o_sem[0])
        for bi in range(N_BATCH)]
    for cp in stage_cps:
        cp.wait()
    for c in range(NB):
        issue_gather(c, c)

    def outer(i, carry):
        for b in range(NB):
            c = i * NB + b
            sbi = b % 2
            @pl.when(c >= 2)
            def _():
                pltpu.make_async_copy(sb[sbi], out_hbm.at[io],
                                      o_sem[sbi]).wait()
            pltpu.make_async_copy(table_hbm.at[io], rows_b[b],
                                  g_sem[b]).wait()
            pltpu.make_async_copy(pos_hbm.at[pl.ds(0, CS)], pos_b[b],
                                  p_sem[b]).wait()

            def add_body(j, jcarry):
                sl = pl.ds(j * LANES, LANES)
                for s in range(CS):
                    pv = pos_b[b][s, sl]
                    for bi in range(N_BATCH):
                        r = bi * CS + s
                        sb[sbi][r, sl] = rows_b[b][r, sl] + pv
                return jcarry
            lax.fori_loop(0, D_MODEL // LANES, add_body, 0)

            pltpu.async_copy(sb[sbi], out_hbm.at[out_lane + c * CS],
                             o_sem[sbi])

            @pl.when(c + NB < N_CHUNKS)
            def _():
                issue_chunk(c + NB, b)
        return carry

    lax.fori_loop(0, N_CHUNKS // NB, outer, 0)

    for sbi in range(2):
        pltpu.make_async_copy(sb[sbi], out_hbm.at[io], o_sem[sbi]).wait()


def kernel(x, table, pos_encoding):
    out = _embed_sc(x.reshape(-1).astype(jnp.int32), table, pos_encoding)
    return out.reshape(N_BATCH, SEQ, D_MODEL)


# speedup vs baseline: 1.2389x; 1.2074x over previous
import functools

import jax
import jax.numpy as jnp
from jax import lax
from jax.experimental import pallas as pl
from jax.experimental.pallas import tpu as pltpu
from jax.experimental.pallas import tpu_sc as plsc

D_MODEL = 1024
N_BATCH = 4
SEQ = 4096
N_TOK = N_BATCH * SEQ
N_WORKERS = 32
S_PER_W = SEQ // N_WORKERS
CS = 4
ROWS = N_BATCH * CS
N_CHUNKS = S_PER_W // CS
N_OUTER = N_CHUNKS // 4
LANES = 16
PER_W = N_BATCH * S_PER_W


@functools.partial(
    pl.kernel,
    mesh=plsc.VectorSubcoreMesh(core_axis_name="c", subcore_axis_name="s"),
    out_type=jax.ShapeDtypeStruct((N_TOK, D_MODEL), jnp.float32),
    scratch_types=(
        [pltpu.VMEM((PER_W,), jnp.int32),
         pltpu.VMEM((PER_W,), jnp.int32),
         pltpu.VMEM((4 * ROWS, D_MODEL), jnp.float32),
         pltpu.VMEM((2 * CS, D_MODEL), jnp.float32),
         pltpu.VMEM((2 * CS, D_MODEL), jnp.float32),
         pltpu.VMEM((ROWS, D_MODEL), jnp.float32),
         pltpu.VMEM((ROWS, D_MODEL), jnp.float32)]
        + [pltpu.SemaphoreType.DMA for _ in range(6)]
    ),
    compiler_params=pltpu.CompilerParams(needs_layout_passes=False),
)
def _embed_sc(x_hbm, table_hbm, pos_hbm, out_hbm,
              stage_v, idx_v, rows_v, posA, posB, sb0, sb1,
              gA, gB, pA, pB, o0, o1):
    pos_h = (posA, posB)
    g_sem = (gA, gB)
    p_sem = (pA, pB)
    sb = (sb0, sb1)
    o_sem = (o0, o1)

    wid = lax.axis_index("s") * 2 + lax.axis_index("c")
    s_base = wid * S_PER_W

    io = lax.iota(jnp.int32, LANES)
    lane_b = lax.shift_right_logical(io, 2)
    lane_s = jnp.bitwise_and(io, 3)
    lane_off = lane_b * S_PER_W + lane_s
    out_lane = lane_b * SEQ + s_base + lane_s

    def issue_pos(i, half):
        pltpu.async_copy(
            pos_hbm.at[pl.ds(s_base + i * (4 * CS) + half * (2 * CS), 2 * CS)],
            pos_h[half], p_sem[half])

    def issue_gather(i, half):
        pltpu.async_copy(
            table_hbm.at[idx_v.at[pl.ds(i * (4 * ROWS) + half * (2 * ROWS),
                                        2 * ROWS)]],
            rows_v.at[pl.ds(half * (2 * ROWS), 2 * ROWS)], g_sem[half])

    def wait_pos(half):
        pltpu.make_async_copy(pos_hbm.at[pl.ds(0, 2 * CS)],
                              pos_h[half], p_sem[half]).wait()

    def wait_gather(half):
        pltpu.make_async_copy(
            table_hbm.at[idx_v.at[pl.ds(0, 2 * ROWS)]],
            rows_v.at[pl.ds(half * (2 * ROWS), 2 * ROWS)],
            g_sem[half]).wait()

    issue_pos(0, 0)
    issue_pos(0, 1)
    stage_cps = [
        pltpu.async_copy(x_hbm.at[pl.ds(bi * SEQ + s_base, S_PER_W)],
                         stage_v.at[pl.ds(bi * S_PER_W, S_PER_W)],
                         o_sem[0])
        for bi in range(N_BATCH)]
    for cp in stage_cps:
        cp.wait()
    for t in range(N_CHUNKS):
        idx_v[pl.ds(t * ROWS, ROWS)] = plsc.load_gather(
            stage_v, [lane_off + t * CS])
    issue_gather(0, 0)
    issue_gather(0, 1)

    def outer(i, carry):
        for b in range(4):
            c = i * 4 + b
            sbi = b % 2
            half = b // 2
            @pl.when(c >= 2)
            def _():
                pltpu.make_async_copy(sb[sbi], out_hbm.at[io],
                                      o_sem[sbi]).wait()
            if b % 2 == 0:
                wait_gather(half)
                wait_pos(half)

            def add_body(j, jcarry):
                sl = pl.ds(j * LANES, LANES)
                for s in range(CS):
                    pv = pos_h[half][(b % 2) * CS + s, sl]
                    for bi in range(N_BATCH):
                        r = bi * CS + s
                        sb[sbi][r, sl] = rows_v[b * ROWS + r, sl] + pv
                return jcarry
            lax.fori_loop(0, D_MODEL // LANES, add_body, 0)

            pltpu.async_copy(sb[sbi], out_hbm.at[out_lane + c * CS],
                             o_sem[sbi])

            if b % 2 == 1:
                @pl.when(i + 1 < N_OUTER)
                def _():
                    issue_gather(i + 1, half)
                    issue_pos(i + 1, half)
        return carry

    lax.fori_loop(0, N_OUTER, outer, 0)

    for sbi in range(2):
        pltpu.make_async_copy(sb[sbi], out_hbm.at[io], o_sem[sbi]).wait()


def kernel(x, table, pos_encoding):
    out = _embed_sc(x.reshape(-1).astype(jnp.int32), table, pos_encoding)
    return out.reshape(N_BATCH, SEQ, D_MODEL)
